# Initial kernel scaffold; baseline (speedup 1.0000x reference)
import functools

import jax
import jax.numpy as jnp
from jax import lax
from jax.experimental import pallas as pl
from jax.experimental.pallas import tpu as pltpu
from jax.experimental.pallas import tpu_sc as plsc

N_NODES = 10000
N_EDGES = 320000
D_FEAT = 128
HALF = D_FEAT // 2

NC = 2
NS = 16
CH = 128
CPT = 160
E_PAD = NS * CPT * CH
SRC_CPT = CPT + 2
ROWS_PT = N_NODES // NS
FB = 125
NFB = ROWS_PT // FB
N_PAD = N_NODES + 16


def _sc_body(xs, srcb, dstb, eps16, out,
             acc, xb, ab, epsv, srcv, dstv, buf0, buf1, g0, g1, s0, s1):
  c = lax.axis_index("c")
  s = lax.axis_index("s")
  row0 = s * ROWS_PT

  pltpu.sync_copy(srcb.at[s], srcv)
  pltpu.sync_copy(dstb.at[s], dstv)
  pltpu.sync_copy(eps16, epsv)

  for b in range(NFB):
    r0 = row0 + b * FB
    pltpu.sync_copy(xs.at[c, pl.ds(r0, FB)], xb)
    pltpu.sync_copy(xb, acc.at[pl.ds(r0, FB)])
  plsc.subcore_barrier()

  pltpu.make_async_copy(xs.at[c].at[srcv.at[0]], buf0, g0).start()
  pltpu.make_async_copy(xs.at[c].at[srcv.at[1]], buf1, g1).start()

  def edge_body(i, carry):
    for b, (buf, gs, ss) in enumerate(((buf0, g0, s0), (buf1, g1, s1))):
      jj = 2 * i + b
      pltpu.make_async_copy(xs.at[c].at[srcv.at[jj]], buf, gs).wait()
      pltpu.async_copy(buf, acc.at[dstv.at[jj]], ss, add=True).wait()
      pltpu.make_async_copy(xs.at[c].at[srcv.at[jj + 2]], buf, gs).start()
    return carry

  lax.fori_loop(0, CPT // 2, edge_body, 0)
  pltpu.make_async_copy(xs.at[c].at[srcv.at[CPT]], buf0, g0).wait()
  pltpu.make_async_copy(xs.at[c].at[srcv.at[CPT + 1]], buf1, g1).wait()
  plsc.subcore_barrier()

  ev = epsv[...]
  for b in range(NFB):
    r0 = row0 + b * FB
    pltpu.sync_copy(acc.at[pl.ds(r0, FB)], ab)
    pltpu.sync_copy(xs.at[c, pl.ds(r0, FB)], xb)

    def row_body(i, carry):
      arow = ab.at[i]
      xrow = xb.at[i]
      for k in range(HALF // 16):
        sl = pl.ds(k * 16, 16)
        arow[sl] = arow[sl] + ev * xrow[sl]
      return carry

    lax.fori_loop(0, FB, row_body, 0)
    pltpu.sync_copy(ab, out.at[c, pl.ds(r0, FB)])


@jax.jit
def kernel(graph, x, eps):
  graph = graph.astype(jnp.int32)
  src = graph[0]
  dst = graph[1]
  srcp = jnp.concatenate(
      [src, jnp.zeros((E_PAD - N_EDGES,), jnp.int32)]).reshape(NS, CPT, CH)
  srcp = jnp.concatenate([srcp, jnp.zeros((NS, 2, CH), jnp.int32)], axis=1)
  dstp = jnp.concatenate(
      [dst, jnp.full((E_PAD - N_EDGES,), N_NODES, jnp.int32)]
  ).reshape(NS, CPT, CH)
  xs = jnp.stack([x[:, :HALF], x[:, HALF:]])
  eps16 = jnp.broadcast_to(eps.astype(jnp.float32), (16,))

  fn = pl.kernel(
      _sc_body,
      out_type=jax.ShapeDtypeStruct((NC, N_NODES, HALF), jnp.float32),
      mesh=plsc.VectorSubcoreMesh(core_axis_name="c", subcore_axis_name="s"),
      scratch_types=[
          pltpu.VMEM_SHARED((N_PAD, HALF), jnp.float32),
          pltpu.VMEM((FB, HALF), jnp.float32),
          pltpu.VMEM((FB, HALF), jnp.float32),
          pltpu.VMEM((16,), jnp.float32),
          pltpu.VMEM((SRC_CPT, CH), jnp.int32),
          pltpu.VMEM((CPT, CH), jnp.int32),
          pltpu.VMEM((CH, HALF), jnp.float32),
          pltpu.VMEM((CH, HALF), jnp.float32),
          pltpu.SemaphoreType.DMA,
          pltpu.SemaphoreType.DMA,
          pltpu.SemaphoreType.DMA,
          pltpu.SemaphoreType.DMA,
      ],
  )
  o = fn(xs, srcp, dstp, eps16)
  return o.transpose(1, 0, 2).reshape(N_NODES, D_FEAT)


# speedup vs baseline: 3.9938x; 3.9938x over previous
import functools

import jax
import jax.numpy as jnp
from jax import lax
from jax.experimental import pallas as pl
from jax.experimental.pallas import tpu as pltpu
from jax.experimental.pallas import tpu_sc as plsc

N_NODES = 10000
N_EDGES = 320000
D_FEAT = 128
HALF = D_FEAT // 2

NC = 2
NS = 16
CH = 128
CPT = 160
E_PAD = NS * CPT * CH
SRC_CPT = CPT + 2
N_RPAD = 10240
ROWS_PT = N_RPAD // NS
FB = 128
NFB = ROWS_PT // FB
N_PAD = N_RPAD


def _sc_body(xs, srcb, dstb, eps16, out,
             acc, xb, ab, epsv, srcv, dstv, buf0, buf1, g0, g1, s0, s1):
  c = lax.axis_index("c")
  s = lax.axis_index("s")
  row0 = s * ROWS_PT

  pltpu.sync_copy(srcb.at[s], srcv)
  pltpu.sync_copy(dstb.at[s], dstv)
  pltpu.sync_copy(eps16, epsv)

  for b in range(NFB):
    r0 = row0 + b * FB
    pltpu.sync_copy(xs.at[c, pl.ds(r0, FB)], xb)
    pltpu.sync_copy(xb, acc.at[pl.ds(r0, FB)])
  plsc.subcore_barrier()

  pltpu.make_async_copy(xs.at[c].at[srcv.at[0]], buf0, g0).start()
  pltpu.make_async_copy(xs.at[c].at[srcv.at[1]], buf1, g1).start()

  def edge_body(i, carry):
    for b, (buf, gs, ss) in enumerate(((buf0, g0, s0), (buf1, g1, s1))):
      jj = 2 * i + b
      pltpu.make_async_copy(xs.at[c].at[srcv.at[jj]], buf, gs).wait()
      pltpu.async_copy(buf, acc.at[dstv.at[jj]], ss, add=True).wait()
      pltpu.make_async_copy(xs.at[c].at[srcv.at[jj + 2]], buf, gs).start()
    return carry

  lax.fori_loop(0, CPT // 2, edge_body, 0)
  pltpu.make_async_copy(xs.at[c].at[srcv.at[CPT]], buf0, g0).wait()
  pltpu.make_async_copy(xs.at[c].at[srcv.at[CPT + 1]], buf1, g1).wait()
  plsc.subcore_barrier()

  ev = epsv[...]
  for b in range(NFB):
    r0 = row0 + b * FB
    pltpu.sync_copy(acc.at[pl.ds(r0, FB)], ab)
    pltpu.sync_copy(xs.at[c, pl.ds(r0, FB)], xb)

    def row_body(i, carry):
      arow = ab.at[i]
      xrow = xb.at[i]
      for k in range(HALF // 16):
        sl = pl.ds(k * 16, 16)
        arow[sl] = arow[sl] + ev * xrow[sl]
      return carry

    lax.fori_loop(0, FB, row_body, 0)
    pltpu.sync_copy(ab, out.at[c, pl.ds(r0, FB)])


@jax.jit
def kernel(graph, x, eps):
  graph = graph.astype(jnp.int32)
  src = graph[0]
  dst = graph[1]
  srcp = jnp.concatenate(
      [src, jnp.zeros((E_PAD - N_EDGES,), jnp.int32)]).reshape(NS, CPT, CH)
  srcp = jnp.concatenate([srcp, jnp.zeros((NS, 2, CH), jnp.int32)], axis=1)
  dstp = jnp.concatenate(
      [dst, jnp.full((E_PAD - N_EDGES,), N_NODES, jnp.int32)]
  ).reshape(NS, CPT, CH)
  xp = jnp.concatenate([x, jnp.zeros((N_RPAD - N_NODES, D_FEAT), x.dtype)])
  xs = jnp.stack([xp[:, :HALF], xp[:, HALF:]])
  eps16 = jnp.broadcast_to(eps.astype(jnp.float32), (16,))

  fn = pl.kernel(
      _sc_body,
      out_type=jax.ShapeDtypeStruct((NC, N_RPAD, HALF), jnp.float32),
      mesh=plsc.VectorSubcoreMesh(core_axis_name="c", subcore_axis_name="s"),
      compiler_params=pltpu.CompilerParams(use_tc_tiling_on_sc=False),
      scratch_types=[
          pltpu.VMEM_SHARED((N_PAD, HALF), jnp.float32),
          pltpu.VMEM((FB, HALF), jnp.float32),
          pltpu.VMEM((FB, HALF), jnp.float32),
          pltpu.VMEM((16,), jnp.float32),
          pltpu.VMEM((SRC_CPT, CH), jnp.int32),
          pltpu.VMEM((CPT, CH), jnp.int32),
          pltpu.VMEM((CH, HALF), jnp.float32),
          pltpu.VMEM((CH, HALF), jnp.float32),
          pltpu.SemaphoreType.DMA,
          pltpu.SemaphoreType.DMA,
          pltpu.SemaphoreType.DMA,
          pltpu.SemaphoreType.DMA,
      ],
  )
  o = fn(xs, srcp, dstp, eps16)
  return o.transpose(1, 0, 2).reshape(N_RPAD, D_FEAT)[:N_NODES]
